# baseline (speedup 1.0000x reference)
import jax
import jax.numpy as jnp
from jax.experimental import pallas as pl
from jax.experimental.pallas import tpu as pltpu

_ROLES = 9
_K = 128
_TM = 4096


def _mm_slice_kernel(x_ref, w_ref, out_ref):
    out_ref[...] = jax.lax.dot_general(
        x_ref[...], w_ref[...],
        dimension_numbers=(((1,), (0,)), ((), ())),
        preferred_element_type=jnp.float32,
    )[:, :_ROLES]


def kernel(x, w_fused_padded):
    b = x.shape[0]
    tm = min(_TM, b)
    steps = pl.cdiv(b, tm)
    return pl.pallas_call(
        _mm_slice_kernel,
        out_shape=jax.ShapeDtypeStruct((b, _ROLES), jnp.float32),
        grid=(steps,),
        in_specs=[
            pl.BlockSpec((tm, _K), lambda i: (i, 0)),
            pl.BlockSpec((_K, _K), lambda i: (0, 0)),
        ],
        out_specs=pl.BlockSpec((tm, _ROLES), lambda i: (i, 0)),
        compiler_params=pltpu.CompilerParams(
            dimension_semantics=("parallel",),
        ),
        cost_estimate=pl.CostEstimate(
            flops=2 * b * _K * _K,
            transcendentals=0,
            bytes_accessed=b * (_K + _ROLES) * 4 + _K * _K * 4,
        ),
    )(x, w_fused_padded)


# speedup vs baseline: 1.1006x; 1.1006x over previous
import jax
import jax.numpy as jnp
from jax.experimental import pallas as pl
from jax.experimental.pallas import tpu as pltpu

_ROLES = 9
_K = 128
_TM = 16384


def _mm_slice_kernel(x_ref, w_ref, out_ref):
    out_ref[...] = jax.lax.dot_general(
        x_ref[...], w_ref[...],
        dimension_numbers=(((1,), (0,)), ((), ())),
        preferred_element_type=jnp.float32,
    )[:, :_ROLES]


def kernel(x, w_fused_padded):
    b = x.shape[0]
    tm = min(_TM, b)
    steps = pl.cdiv(b, tm)
    return pl.pallas_call(
        _mm_slice_kernel,
        out_shape=jax.ShapeDtypeStruct((b, _ROLES), jnp.float32),
        grid=(steps,),
        in_specs=[
            pl.BlockSpec((tm, _K), lambda i: (i, 0)),
            pl.BlockSpec((_K, _K), lambda i: (0, 0)),
        ],
        out_specs=pl.BlockSpec((tm, _ROLES), lambda i: (i, 0)),
        compiler_params=pltpu.CompilerParams(
            dimension_semantics=("parallel",),
        ),
        cost_estimate=pl.CostEstimate(
            flops=2 * b * _K * _K,
            transcendentals=0,
            bytes_accessed=b * (_K + _ROLES) * 4 + _K * _K * 4,
        ),
    )(x, w_fused_padded)


# speedup vs baseline: 1.1025x; 1.0018x over previous
import jax
import jax.numpy as jnp
from jax.experimental import pallas as pl
from jax.experimental.pallas import tpu as pltpu

_ROLES = 9
_K = 128
_TM = 16384


def _mm_slice_kernel(x_ref, w_ref, out_ref):
    out_ref[...] = jax.lax.dot_general(
        x_ref[...], w_ref[...],
        dimension_numbers=(((1,), (0,)), ((), ())),
        preferred_element_type=jnp.float32,
    )[:, :_ROLES]


def kernel(x, w_fused_padded):
    b = x.shape[0]
    tm = min(_TM, b)
    steps = pl.cdiv(b, tm)
    return pl.pallas_call(
        _mm_slice_kernel,
        out_shape=jax.ShapeDtypeStruct((b, _ROLES), jnp.float32),
        grid=(steps,),
        in_specs=[
            pl.BlockSpec((tm, _K), lambda i: (i, 0)),
            pl.BlockSpec((_K, _K), lambda i: (0, 0)),
        ],
        out_specs=pl.BlockSpec((tm, _ROLES), lambda i: (i, 0)),
        compiler_params=pltpu.CompilerParams(
            dimension_semantics=("arbitrary",),
        ),
        cost_estimate=pl.CostEstimate(
            flops=2 * b * _K * _K,
            transcendentals=0,
            bytes_accessed=b * (_K + _ROLES) * 4 + _K * _K * 4,
        ),
    )(x, w_fused_padded)
